# Initial kernel scaffold; baseline (speedup 1.0000x reference)
import jax
import jax.numpy as jnp
from jax.experimental import pallas as pl


def kernel(atom_features, edge_index, r, params):
    raise NotImplementedError("write your pallas kernel here")


# speedup vs baseline: 5.2682x; 5.2682x over previous
import functools

import jax
import jax.numpy as jnp
from jax import lax
from jax.experimental import pallas as pl
from jax.experimental.pallas import tpu as pltpu
from jax.experimental.pallas import tpu_sc as plsc

NC = 2
NS = 16
NW = NC * NS
K = 128
D = 32
EPS = 1e-5


def _make_deg_kernel(nsc, epw):
  rsub = nsc // NS
  nchunk = epw // K
  mesh = plsc.VectorSubcoreMesh(core_axis_name="c", subcore_axis_name="s")

  @functools.partial(
      pl.kernel,
      out_type=(jax.ShapeDtypeStruct((NC * nsc,), jnp.float32),
                jax.ShapeDtypeStruct((NC * nsc,), jnp.float32)),
      mesh=mesh,
      scratch_types=[
          pltpu.VMEM_SHARED((nsc,), jnp.float32),
          pltpu.VMEM_SHARED((nsc,), jnp.float32),
          pltpu.VMEM((K,), jnp.int32),
          pltpu.VMEM((K,), jnp.int32),
          pltpu.VMEM((K,), jnp.float32),
      ],
  )
  def deg_kernel(srcf, dstf, maskf, zeros1, dego, degi, dego_sh, degi_sh,
                 src_v, dst_v, mask_v):
    c = lax.axis_index("c")
    s = lax.axis_index("s")
    w = s * NC + c
    sl = pl.ds(s * rsub, rsub)
    pltpu.sync_copy(zeros1.at[sl], dego_sh.at[sl])
    pltpu.sync_copy(zeros1.at[sl], degi_sh.at[sl])
    plsc.subcore_barrier()

    ebase = w * epw

    @pl.loop(0, nchunk)
    def _chunk(j):
      base = ebase + j * K
      pltpu.sync_copy(srcf.at[pl.ds(base, K)], src_v)
      pltpu.sync_copy(dstf.at[pl.ds(base, K)], dst_v)
      pltpu.sync_copy(maskf.at[pl.ds(base, K)], mask_v)
      pltpu.sync_copy(mask_v, dego_sh.at[src_v], add=True)
      pltpu.sync_copy(mask_v, degi_sh.at[dst_v], add=True)

    plsc.subcore_barrier()
    osl = pl.ds(c * nsc + s * rsub, rsub)
    pltpu.sync_copy(dego_sh.at[sl], dego.at[osl])
    pltpu.sync_copy(degi_sh.at[sl], degi.at[osl])

  return deg_kernel


def _make_spmm_kernel(nsc, epw):
  rsub = nsc // NS
  nchunk = epw // K
  mesh = plsc.VectorSubcoreMesh(core_axis_name="c", subcore_axis_name="s")

  @functools.partial(
      pl.kernel,
      out_type=jax.ShapeDtypeStruct((NC, nsc, D), jnp.float32),
      mesh=mesh,
      scratch_types=[
          pltpu.VMEM_SHARED((nsc, D), jnp.float32),
          pltpu.VMEM((K,), jnp.int32),
          pltpu.VMEM((K,), jnp.int32),
          pltpu.VMEM((K,), jnp.float32),
          pltpu.VMEM((K, D), jnp.float32),
          pltpu.SemaphoreType.DMA,
      ],
      compiler_params=pltpu.CompilerParams(use_tc_tiling_on_sc=False),
  )
  def spmm_kernel(table, srcf, dstf, ewf, zeros, out, agg_sh,
                  src_v, dst_v, ew_v, rows_v, sem):
    c = lax.axis_index("c")
    s = lax.axis_index("s")
    w = s * NC + c
    sl = pl.ds(s * rsub, rsub)
    pltpu.sync_copy(zeros.at[sl], agg_sh.at[sl])
    plsc.subcore_barrier()

    ebase = w * epw

    @pl.loop(0, nchunk)
    def _chunk(j):
      base = ebase + j * K
      pltpu.sync_copy(srcf.at[pl.ds(base, K)], src_v)
      pltpu.sync_copy(dstf.at[pl.ds(base, K)], dst_v)
      pltpu.sync_copy(ewf.at[pl.ds(base, K)], ew_v)
      pltpu.async_copy(table.at[src_v], rows_v, sem).wait()

      @pl.loop(0, K // 16)
      def _mul(g):
        wv = ew_v[pl.ds(g * 16, 16)]
        for u in range(16):
          e = g * 16 + u
          wt = wv[u]
          rows_v[e, pl.ds(0, 16)] = rows_v[e, pl.ds(0, 16)] * wt
          rows_v[e, pl.ds(16, 16)] = rows_v[e, pl.ds(16, 16)] * wt

      pltpu.sync_copy(rows_v, agg_sh.at[dst_v], add=True)

    plsc.subcore_barrier()
    pltpu.sync_copy(agg_sh.at[sl], out.at[c, sl])

  return spmm_kernel


def _ew_body(r_ref, m_ref, o_ref):
  rr = r_ref[...]
  rn2 = jnp.sum(rr * rr, axis=0, keepdims=True)
  o_ref[...] = jnp.exp(rn2 * (-1.0 / 16.0)) * m_ref[...]


def _ew_kernel(r_t, maskf, epad):
  nb = 8
  be = epad // nb
  return pl.pallas_call(
      _ew_body,
      grid=(nb,),
      in_specs=[
          pl.BlockSpec((3, be), lambda i: (0, i)),
          pl.BlockSpec((1, be), lambda i: (0, i)),
      ],
      out_specs=pl.BlockSpec((1, be), lambda i: (0, i)),
      out_shape=jax.ShapeDtypeStruct((1, epad), jnp.float32),
  )(r_t, maskf)


def _atomstats_body(a_ref, s_ref, q_ref):
  a = a_ref[...]

  @pl.when(pl.program_id(0) == 0)
  def _():
    s_ref[...] = jnp.zeros_like(s_ref)
    q_ref[...] = jnp.zeros_like(q_ref)

  s_ref[...] += jnp.sum(a, axis=0, keepdims=True)
  q_ref[...] += jnp.sum(a * a, axis=0, keepdims=True)


def _atomstats_kernel(atom, n, bn):
  nb = n // bn
  return pl.pallas_call(
      _atomstats_body,
      grid=(nb,),
      in_specs=[pl.BlockSpec((bn, 1), lambda i: (i, 0))],
      out_specs=[
          pl.BlockSpec((1, 1), lambda i: (0, 0)),
          pl.BlockSpec((1, 1), lambda i: (0, 0)),
      ],
      out_shape=[
          jax.ShapeDtypeStruct((1, 1), jnp.float32),
          jax.ShapeDtypeStruct((1, 1), jnp.float32),
      ],
  )(atom)


def _emb_body(a_ref, w_ref, b_ref, f_ref, s_ref, q_ref):
  h = jax.nn.relu(a_ref[...] * w_ref[...] + b_ref[...])
  f_ref[...] = h

  @pl.when(pl.program_id(0) == 0)
  def _():
    s_ref[...] = jnp.zeros_like(s_ref)
    q_ref[...] = jnp.zeros_like(q_ref)

  s_ref[...] += jnp.sum(h, axis=0, keepdims=True)
  q_ref[...] += jnp.sum(h * h, axis=0, keepdims=True)


def _emb_kernel(atom, aff_a, aff_c, n, bn):
  nb = n // bn
  return pl.pallas_call(
      _emb_body,
      grid=(nb,),
      in_specs=[
          pl.BlockSpec((bn, 1), lambda i: (i, 0)),
          pl.BlockSpec((1, D), lambda i: (0, 0)),
          pl.BlockSpec((1, D), lambda i: (0, 0)),
      ],
      out_specs=[
          pl.BlockSpec((bn, D), lambda i: (i, 0)),
          pl.BlockSpec((1, D), lambda i: (0, 0)),
          pl.BlockSpec((1, D), lambda i: (0, 0)),
      ],
      out_shape=[
          jax.ShapeDtypeStruct((n, D), jnp.float32),
          jax.ShapeDtypeStruct((1, D), jnp.float32),
          jax.ShapeDtypeStruct((1, D), jnp.float32),
      ],
  )(atom, aff_a, aff_c)


def _make_aggstats_body(n, bn):
  def body(p_ref, di_ref, b_ref, f_ref, s_ref, q_ref):
    deg = di_ref[:, 0:1] + di_ref[:, 1:2]
    dr = lax.rsqrt(jnp.maximum(deg, 1.0))
    f = (p_ref[0] + p_ref[1]) * dr + b_ref[...]
    rid = pl.program_id(0) * bn + lax.broadcasted_iota(jnp.int32, (bn, 1), 0)
    f = jnp.where(rid < n, f, 0.0)
    f_ref[...] = f

    @pl.when(pl.program_id(0) == 0)
    def _():
      s_ref[...] = jnp.zeros_like(s_ref)
      q_ref[...] = jnp.zeros_like(q_ref)

    s_ref[...] += jnp.sum(f, axis=0, keepdims=True)
    q_ref[...] += jnp.sum(f * f, axis=0, keepdims=True)
  return body


def _aggstats_kernel(partials, din_t, b, n, nsc, bn):
  nb = nsc // bn
  return pl.pallas_call(
      _make_aggstats_body(n, bn),
      grid=(nb,),
      in_specs=[
          pl.BlockSpec((NC, bn, D), lambda i: (0, i, 0)),
          pl.BlockSpec((bn, NC), lambda i: (i, 0)),
          pl.BlockSpec((1, D), lambda i: (0, 0)),
      ],
      out_specs=[
          pl.BlockSpec((bn, D), lambda i: (i, 0)),
          pl.BlockSpec((1, D), lambda i: (0, 0)),
          pl.BlockSpec((1, D), lambda i: (0, 0)),
      ],
      out_shape=[
          jax.ShapeDtypeStruct((nsc, D), jnp.float32),
          jax.ShapeDtypeStruct((1, D), jnp.float32),
          jax.ShapeDtypeStruct((1, D), jnp.float32),
      ],
  )(partials, din_t, b)


def _make_table_body(nblk):
  def body(*refs):
    do_ref = refs[0]
    f_refs = refs[1:1 + nblk]
    sc_refs = refs[1 + nblk:1 + 2 * nblk]
    sh_refs = refs[1 + 2 * nblk:1 + 3 * nblk]
    w_refs = refs[1 + 3 * nblk:1 + 4 * nblk]
    t_ref = refs[-1]
    deg = do_ref[:, 0:1] + do_ref[:, 1:2]
    dr = lax.rsqrt(jnp.maximum(deg, 1.0))
    acc = None
    for j in range(nblk):
      z = jax.nn.relu(f_refs[j][...] * sc_refs[j][...] + sh_refs[j][...]) * dr
      part = jnp.dot(z, w_refs[j][...], preferred_element_type=jnp.float32)
      acc = part if acc is None else acc + part
    t_ref[...] = acc
  return body


def _table_kernel(dout_t, blocks, scales, shifts, ws, nsc, bn):
  nb = nsc // bn
  nblk = len(blocks)
  in_specs = [pl.BlockSpec((bn, NC), lambda i: (i, 0))]
  in_specs += [pl.BlockSpec((bn, D), lambda i: (i, 0))] * nblk
  in_specs += [pl.BlockSpec((1, D), lambda i: (0, 0))] * (2 * nblk)
  in_specs += [pl.BlockSpec((D, D), lambda i: (0, 0))] * nblk
  return pl.pallas_call(
      _make_table_body(nblk),
      grid=(nb,),
      in_specs=in_specs,
      out_specs=pl.BlockSpec((bn, D), lambda i: (i, 0)),
      out_shape=jax.ShapeDtypeStruct((nsc, D), jnp.float32),
  )(dout_t, *blocks, *scales, *shifts, *ws)


def _make_readout_body(nblk, n, bn):
  def body(*refs):
    f_refs = refs[0:nblk]
    a_refs = refs[nblk:2 * nblk]
    c_refs = refs[2 * nblk:3 * nblk]
    s_refs = refs[3 * nblk:]

    @pl.when(pl.program_id(0) == 0)
    def _():
      for s_ref in s_refs:
        s_ref[...] = jnp.zeros_like(s_ref)

    rid = pl.program_id(0) * bn + lax.broadcasted_iota(jnp.int32, (bn, 1), 0)
    valid = rid < n
    for j in range(nblk):
      z = jax.nn.relu(f_refs[j][...] * a_refs[j][...] + c_refs[j][...])
      z = jnp.where(valid, z, 0.0)
      s_refs[j][...] += jnp.sum(z, axis=0, keepdims=True)
  return body


def _readout_kernel(blocks, avecs, cvecs, n, nsc, bn):
  nb = nsc // bn
  nblk = len(blocks)
  in_specs = [pl.BlockSpec((bn, D), lambda i: (i, 0))] * nblk
  in_specs += [pl.BlockSpec((1, D), lambda i: (0, 0))] * (2 * nblk)
  return pl.pallas_call(
      _make_readout_body(nblk, n, bn),
      grid=(nb,),
      in_specs=in_specs,
      out_specs=[pl.BlockSpec((1, D), lambda i: (0, 0))] * nblk,
      out_shape=[jax.ShapeDtypeStruct((1, D), jnp.float32)] * nblk,
  )(*blocks, *avecs, *cvecs)


def _affine(ssum, ssq, g, b, n):
  m = ssum / n
  v = ssq / n - m * m
  inv = lax.rsqrt(v + EPS)
  scale = g.reshape(1, D) * inv
  shift = b.reshape(1, D) - m * scale
  return scale, shift


def kernel(atom_features, edge_index, r, params):
  n = atom_features.shape[0]
  e = edge_index.shape[1]
  rsub = ((-(-n // NS) + K - 1) // K) * K
  nsc = rsub * NS
  epw = -(-e // (NW * K)) * K
  epad = epw * NW
  pad = epad - e

  srcf = jnp.pad(edge_index[0], (0, pad))
  dstf = jnp.pad(edge_index[1], (0, pad))
  mask = jnp.pad(jnp.ones((e,), jnp.float32), (0, pad))
  r_t = jnp.pad(r, ((0, pad), (0, 0))).T

  ewf = _ew_kernel(r_t, mask.reshape(1, epad), epad).reshape(epad)

  zeros1 = jnp.zeros((nsc,), jnp.float32)
  dego, degi = _make_deg_kernel(nsc, epw)(srcf, dstf, mask, zeros1)
  dout_t = dego.reshape(NC, nsc).T
  din_t = degi.reshape(NC, nsc).T

  bn_n = next(c for c in range(min(n, 10000), 0, -1) if n % c == 0 and c % 8 == 0)
  bn_s = next(c for c in range(min(nsc, 6400), 0, -1) if nsc % c == 0 and c % 8 == 0)
  p = params

  sa, qa = _atomstats_kernel(atom_features, n, bn_n)
  m_a = sa / n
  v_a = qa / n - m_a * m_a
  w0 = p['emb_W']
  m_h = m_a * w0 + p['emb_b'].reshape(1, D)
  inv0 = lax.rsqrt(v_a * w0 * w0 + EPS)
  g0 = p['bn0_g'].reshape(1, D)
  aff_a = g0 * inv0 * w0
  aff_c = g0 * inv0 * (p['emb_b'].reshape(1, D) - m_h) + p['bn0_b'].reshape(1, D)
  f0, s0, q0 = _emb_kernel(atom_features, aff_a, aff_c, n, bn_n)
  f0 = jnp.pad(f0, ((0, nsc - n), (0, 0)))

  zeros_nd = jnp.zeros((nsc, D), jnp.float32)
  spmm = _make_spmm_kernel(nsc, epw)

  blocks = [f0]
  sums = [s0]
  sqs = [q0]

  for li, lp in enumerate(p['layers']):
    nblk = li + 1
    scales = []
    shifts = []
    for j in range(nblk):
      g = lp['bn_g'][j * D:(j + 1) * D]
      bb = lp['bn_b'][j * D:(j + 1) * D]
      s_, h_ = _affine(sums[j], sqs[j], g, bb, n)
      scales.append(s_)
      shifts.append(h_)
    ws = [lp['W'][j * D:(j + 1) * D, :] for j in range(nblk)]
    table = _table_kernel(dout_t, blocks, scales, shifts, ws, nsc, bn_s)
    partials = spmm(table, srcf, dstf, ewf, zeros_nd)
    f, s, q = _aggstats_kernel(partials, din_t, lp['b'].reshape(1, D), n, nsc,
                               bn_s)
    blocks.append(f)
    sums.append(s)
    sqs.append(q)

  nblk = len(blocks)
  avecs = []
  cvecs = []
  for j in range(nblk):
    g = p['bnf_g'][j * D:(j + 1) * D]
    bb = p['bnf_b'][j * D:(j + 1) * D]
    a_, c_ = _affine(sums[j], sqs[j], g, bb, n)
    avecs.append(a_)
    cvecs.append(c_)
  colsums = _readout_kernel(blocks, avecs, cvecs, n, nsc, bn_s)
  pooled = jnp.concatenate([cs.reshape(D) for cs in colsums]) / n
  out = pooled @ p['fc_W'] + p['fc_b']
  return jnp.squeeze(out)
